# Initial kernel scaffold; baseline (speedup 1.0000x reference)
import jax
import jax.numpy as jnp
from jax.experimental import pallas as pl


def kernel(x, adj, edge_weight, W, b):
    raise NotImplementedError("write your pallas kernel here")


# speedup vs baseline: 4.7022x; 4.7022x over previous
import functools

import jax
import jax.numpy as jnp
from jax.experimental import pallas as pl


def _gcn_kernel(adj_ref, diag_ref, x_ref, xb_ref, w_ref, b_ref, out_ref, *, blk):
    a = adj_ref[:, :]
    xfull = x_ref[:, :]

    ax = jnp.dot(a, xfull, preferred_element_type=jnp.float32)
    w = w_ref[:, :]
    s = jnp.dot(ax, w, preferred_element_type=jnp.float32)

    deg = jnp.sum(a, axis=1, keepdims=True)

    dsub = diag_ref[:, :]
    rows = jax.lax.broadcasted_iota(jnp.int32, (blk, blk), 0)
    cols = jax.lax.broadcasted_iota(jnp.int32, (blk, blk), 1)
    d = jnp.sum(jnp.where(rows == cols, dsub, 0.0), axis=1, keepdims=True)

    xw = jnp.dot(xb_ref[:, :], w, preferred_element_type=jnp.float32)

    inv = jnp.where(deg > 0.0, 1.0 / deg, 0.0)
    out_ref[:, :] = xw - inv * (2.0 * s * xw - 2.0 * d * xw * xw) - b_ref[:, :]


def kernel(x, adj, edge_weight, W, b):
    del edge_weight
    n, d_in = x.shape
    d_out = W.shape[1]
    blk = 512
    grid = (pl.cdiv(n, blk),)
    out = pl.pallas_call(
        functools.partial(_gcn_kernel, blk=blk),
        grid=grid,
        in_specs=[
            pl.BlockSpec((blk, n), lambda i: (i, 0)),
            pl.BlockSpec((blk, blk), lambda i: (i, i)),
            pl.BlockSpec((n, d_in), lambda i: (0, 0)),
            pl.BlockSpec((blk, d_in), lambda i: (i, 0)),
            pl.BlockSpec((d_in, d_out), lambda i: (0, 0)),
            pl.BlockSpec((1, d_out), lambda i: (0, 0)),
        ],
        out_specs=pl.BlockSpec((blk, d_out), lambda i: (i, 0)),
        out_shape=jax.ShapeDtypeStruct((n, d_out), jnp.float32),
    )(adj, adj, x, x, W, b.reshape(1, d_out))
    return out
